# baseline (speedup 1.0000x reference)
import functools

import jax
import jax.numpy as jnp
from jax import lax
from jax.experimental import pallas as pl
from jax.experimental.pallas import tpu as pltpu

_EPS = 1e-4
_RATIO_BALANCE = 1.0


def _loss_body(idx_ref, outputs_ref, label_ref, feat_ref, u_ref, w_ref,
               sl_ref, ps_ref, loss_ref, mv_ref, macc_ref, zacc_ref,
               xent_ref, mse_ref, avg_ref, cnt_ref, se_ref,
               *, blk, nb, num_classes, num_examp, enc):
    i = pl.program_id(0)
    c = num_classes

    @pl.when(i == 0)
    def _init():
        seg = (lax.broadcasted_iota(jnp.int32, (c, num_examp), 0)
               == sl_ref[...]).astype(jnp.float32)
        seg_sum = jnp.dot(seg, ps_ref[...],
                          preferred_element_type=jnp.float32)
        counts = jnp.sum(seg, axis=1, keepdims=True)
        mv = seg_sum / counts
        mv_ref[...] = mv * lax.rsqrt(jnp.sum(mv * mv, axis=1, keepdims=True))
        macc_ref[...] = jnp.zeros_like(macc_ref)
        zacc_ref[...] = jnp.zeros_like(zacc_ref)
        xent_ref[...] = jnp.zeros_like(xent_ref)
        mse_ref[...] = jnp.zeros_like(mse_ref)
        avg_ref[...] = jnp.zeros_like(avg_ref)
        cnt_ref[...] = jnp.zeros_like(cnt_ref)
        se_ref[...] = jnp.zeros_like(se_ref)

    o = outputs_ref[...]
    lab = label_ref[...]
    feat = feat_ref[...]
    idx = idx_ref[...]

    oh = (idx == lax.broadcasted_iota(jnp.int32, (blk, num_examp), 1)
          ).astype(jnp.float32)
    u_g = jnp.sum(oh * u_ref[...], axis=1, keepdims=True)
    w_g = jnp.sum(oh * w_ref[...], axis=1, keepdims=True)

    inv_n = lax.rsqrt(jnp.sum(feat * feat, axis=1, keepdims=True))
    sim = lax.dot_general(feat, mv_ref[...], (((1,), (1,)), ((), ())),
                          preferred_element_type=jnp.float32)
    x = inv_n * jnp.maximum(sim * lab, 0.0)

    m = jnp.max(o, axis=1, keepdims=True)
    e = jnp.exp(o - m)
    pred = e / jnp.sum(e, axis=1, keepdims=True)
    ul = u_g * lab
    predc = jnp.clip(pred + (1.0 - w_g) * ul, _EPS, 1.0)
    xent_ref[...] += jnp.sum(x * jnp.log(predc), axis=0, keepdims=True)

    iota_c = lax.broadcasted_iota(jnp.int32, (blk, c), 1)
    first = jnp.min(jnp.where(o == m, iota_c, c), axis=1, keepdims=True)
    d = (iota_c == first).astype(jnp.float32) + w_g * ul - lab
    mse_ref[...] += jnp.sum(d * d, axis=0, keepdims=True)

    avg_ref[...] += jnp.sum(predc, axis=0, keepdims=True)
    cnt_ref[...] += jnp.sum(oh, axis=0, keepdims=True)

    s_col = jnp.sum(o * lab, axis=1, keepdims=True)
    se_ref[...] += jnp.sum(oh * s_col, axis=0, keepdims=True)
    m_i = jnp.max(s_col)
    z_i = jnp.sum(jnp.exp(s_col - m_i))
    lanes = lax.broadcasted_iota(jnp.int32, (1, 128), 1)
    macc_ref[...] += jnp.where(lanes == i, m_i, 0.0)
    zacc_ref[...] += jnp.where(lanes == i, z_i, 0.0)

    @pl.when(i == nb - 1)
    def _fin():
        btot = float(nb * blk)
        live = lanes < nb
        m_row = macc_ref[...]
        big_m = jnp.max(jnp.where(live, m_row, -3.0e38), axis=1,
                        keepdims=True)
        z = jnp.sum(jnp.where(live, zacc_ref[...] * jnp.exp(m_row - big_m),
                              0.0), axis=1, keepdims=True)
        lse_s = big_m + jnp.log(z)

        eu = jnp.exp(-u_ref[...])
        n_e = cnt_ref[...]
        a2 = jnp.sum(n_e * eu, axis=1, keepdims=True)
        a3 = jnp.sum(n_e * eu * (-u_ref[...]), axis=1, keepdims=True)
        a4 = jnp.sum(eu * se_ref[...], axis=1, keepdims=True)
        kl = (a3 / a2 - jnp.log(a2) - a4 / a2 + lse_s) / btot

        avg = jnp.clip(avg_ref[...] / btot, _EPS, 1.0)
        bal = -jnp.sum(jnp.log(avg), axis=1, keepdims=True) / float(c)

        loss1 = -jnp.sum(xent_ref[...], axis=1, keepdims=True)
        mse = jnp.sum(mse_ref[...], axis=1, keepdims=True)
        loss_ref[...] = (loss1 + mse) / btot + kl + _RATIO_BALANCE * bal


def kernel(index, outputs, label, out, flag, epoch, train_acc_cater, u,
           prevSimilarity, weight, sample_labels):
    del flag, epoch, train_acc_cater
    b, c = outputs.shape
    enc = out.shape[1]
    num_examp = u.shape[0]
    blk = 2048
    nb = b // blk

    idx2d = index.astype(jnp.int32).reshape(b, 1)
    u_row = u.reshape(1, num_examp)
    w_row = weight.reshape(1, num_examp)
    sl_row = sample_labels.astype(jnp.int32).reshape(1, num_examp)

    body = functools.partial(_loss_body, blk=blk, nb=nb, num_classes=c,
                             num_examp=num_examp, enc=enc)
    result = pl.pallas_call(
        body,
        grid=(nb,),
        in_specs=[
            pl.BlockSpec((blk, 1), lambda i: (i, 0)),
            pl.BlockSpec((blk, c), lambda i: (i, 0)),
            pl.BlockSpec((blk, c), lambda i: (i, 0)),
            pl.BlockSpec((blk, enc), lambda i: (i, 0)),
            pl.BlockSpec((1, num_examp), lambda i: (0, 0)),
            pl.BlockSpec((1, num_examp), lambda i: (0, 0)),
            pl.BlockSpec((1, num_examp), lambda i: (0, 0)),
            pl.BlockSpec((c, enc), lambda i: (0, 0)),
        ],
        out_specs=pl.BlockSpec((1, 1), lambda i: (0, 0)),
        out_shape=jax.ShapeDtypeStruct((1, 1), jnp.float32),
        scratch_shapes=[
            pltpu.VMEM((c, enc), jnp.float32),
            pltpu.VMEM((1, 128), jnp.float32),
            pltpu.VMEM((1, 128), jnp.float32),
            pltpu.VMEM((1, c), jnp.float32),
            pltpu.VMEM((1, c), jnp.float32),
            pltpu.VMEM((1, c), jnp.float32),
            pltpu.VMEM((1, num_examp), jnp.float32),
            pltpu.VMEM((1, num_examp), jnp.float32),
        ],
    )(idx2d, outputs, label, out, u_row, w_row, sl_row, prevSimilarity)
    return result[0, 0]


# speedup vs baseline: 1.0432x; 1.0432x over previous
import functools

import jax
import jax.numpy as jnp
from jax import lax
from jax.experimental import pallas as pl
from jax.experimental.pallas import tpu as pltpu

_EPS = 1e-4
_RATIO_BALANCE = 1.0


def _loss_body(idx_ref, outputs_ref, label_ref, feat_ref, u_ref, w_ref,
               sl_ref, ps_ref, loss_ref, mv_ref, macc_ref, zacc_ref,
               xent_ref, mse_ref, avg_ref, cnt_ref, se_ref,
               *, blk, nb, num_classes, num_examp, enc):
    i = pl.program_id(0)
    c = num_classes

    @pl.when(i == 0)
    def _init():
        seg = (lax.broadcasted_iota(jnp.int32, (c, num_examp), 0)
               == sl_ref[...]).astype(jnp.float32)
        seg_sum = jnp.dot(seg, ps_ref[...],
                          preferred_element_type=jnp.float32)
        counts = jnp.sum(seg, axis=1, keepdims=True)
        mv = seg_sum / counts
        mv_ref[...] = mv * lax.rsqrt(jnp.sum(mv * mv, axis=1, keepdims=True))
        macc_ref[...] = jnp.zeros_like(macc_ref)
        zacc_ref[...] = jnp.zeros_like(zacc_ref)
        xent_ref[...] = jnp.zeros_like(xent_ref)
        mse_ref[...] = jnp.zeros_like(mse_ref)
        avg_ref[...] = jnp.zeros_like(avg_ref)
        cnt_ref[...] = jnp.zeros_like(cnt_ref)
        se_ref[...] = jnp.zeros_like(se_ref)

    o = outputs_ref[...]
    lab = label_ref[...]
    feat = feat_ref[...]
    idx = idx_ref[...]

    oh = (idx == lax.broadcasted_iota(jnp.int32, (blk, num_examp), 1)
          ).astype(jnp.float32)
    u_g = jnp.sum(oh * u_ref[...], axis=1, keepdims=True)
    w_g = jnp.sum(oh * w_ref[...], axis=1, keepdims=True)

    inv_n = lax.rsqrt(jnp.sum(feat * feat, axis=1, keepdims=True))
    sim = lax.dot_general(feat, mv_ref[...], (((1,), (1,)), ((), ())),
                          preferred_element_type=jnp.float32)
    x = inv_n * jnp.maximum(sim * lab, 0.0)

    m = jnp.max(o, axis=1, keepdims=True)
    e = jnp.exp(o - m)
    pred = e / jnp.sum(e, axis=1, keepdims=True)
    ul = u_g * lab
    predc = jnp.clip(pred + (1.0 - w_g) * ul, _EPS, 1.0)
    xent_ref[...] += jnp.sum(x * jnp.log(predc), axis=0, keepdims=True)

    iota_c = lax.broadcasted_iota(jnp.int32, (blk, c), 1)
    first = jnp.min(jnp.where(o == m, iota_c, c), axis=1, keepdims=True)
    d = (iota_c == first).astype(jnp.float32) + w_g * ul - lab
    mse_ref[...] += jnp.sum(d * d, axis=0, keepdims=True)

    avg_ref[...] += jnp.sum(predc, axis=0, keepdims=True)
    cnt_ref[...] += jnp.sum(oh, axis=0, keepdims=True)

    s_col = jnp.sum(o * lab, axis=1, keepdims=True)
    se_ref[...] += jnp.sum(oh * s_col, axis=0, keepdims=True)
    m_i = jnp.max(s_col)
    z_i = jnp.sum(jnp.exp(s_col - m_i))
    lanes = lax.broadcasted_iota(jnp.int32, (1, 128), 1)
    macc_ref[...] += jnp.where(lanes == i, m_i, 0.0)
    zacc_ref[...] += jnp.where(lanes == i, z_i, 0.0)

    @pl.when(i == nb - 1)
    def _fin():
        btot = float(nb * blk)
        live = lanes < nb
        m_row = macc_ref[...]
        big_m = jnp.max(jnp.where(live, m_row, -3.0e38), axis=1,
                        keepdims=True)
        z = jnp.sum(jnp.where(live, zacc_ref[...] * jnp.exp(m_row - big_m),
                              0.0), axis=1, keepdims=True)
        lse_s = big_m + jnp.log(z)

        eu = jnp.exp(-u_ref[...])
        n_e = cnt_ref[...]
        a2 = jnp.sum(n_e * eu, axis=1, keepdims=True)
        a3 = jnp.sum(n_e * eu * (-u_ref[...]), axis=1, keepdims=True)
        a4 = jnp.sum(eu * se_ref[...], axis=1, keepdims=True)
        kl = (a3 / a2 - jnp.log(a2) - a4 / a2 + lse_s) / btot

        avg = jnp.clip(avg_ref[...] / btot, _EPS, 1.0)
        bal = -jnp.sum(jnp.log(avg), axis=1, keepdims=True) / float(c)

        loss1 = -jnp.sum(xent_ref[...], axis=1, keepdims=True)
        mse = jnp.sum(mse_ref[...], axis=1, keepdims=True)
        loss_ref[...] = (loss1 + mse) / btot + kl + _RATIO_BALANCE * bal


def kernel(index, outputs, label, out, flag, epoch, train_acc_cater, u,
           prevSimilarity, weight, sample_labels):
    del flag, epoch, train_acc_cater
    b, c = outputs.shape
    enc = out.shape[1]
    num_examp = u.shape[0]
    blk = 1024
    nb = b // blk

    idx2d = index.astype(jnp.int32).reshape(b, 1)
    u_row = u.reshape(1, num_examp)
    w_row = weight.reshape(1, num_examp)
    sl_row = sample_labels.astype(jnp.int32).reshape(1, num_examp)

    body = functools.partial(_loss_body, blk=blk, nb=nb, num_classes=c,
                             num_examp=num_examp, enc=enc)
    result = pl.pallas_call(
        body,
        grid=(nb,),
        in_specs=[
            pl.BlockSpec((blk, 1), lambda i: (i, 0)),
            pl.BlockSpec((blk, c), lambda i: (i, 0)),
            pl.BlockSpec((blk, c), lambda i: (i, 0)),
            pl.BlockSpec((blk, enc), lambda i: (i, 0)),
            pl.BlockSpec((1, num_examp), lambda i: (0, 0)),
            pl.BlockSpec((1, num_examp), lambda i: (0, 0)),
            pl.BlockSpec((1, num_examp), lambda i: (0, 0)),
            pl.BlockSpec((c, enc), lambda i: (0, 0)),
        ],
        out_specs=pl.BlockSpec((1, 1), lambda i: (0, 0)),
        out_shape=jax.ShapeDtypeStruct((1, 1), jnp.float32),
        scratch_shapes=[
            pltpu.VMEM((c, enc), jnp.float32),
            pltpu.VMEM((1, 128), jnp.float32),
            pltpu.VMEM((1, 128), jnp.float32),
            pltpu.VMEM((1, c), jnp.float32),
            pltpu.VMEM((1, c), jnp.float32),
            pltpu.VMEM((1, c), jnp.float32),
            pltpu.VMEM((1, num_examp), jnp.float32),
            pltpu.VMEM((1, num_examp), jnp.float32),
        ],
    )(idx2d, outputs, label, out, u_row, w_row, sl_row, prevSimilarity)
    return result[0, 0]
